# Initial kernel scaffold; baseline (speedup 1.0000x reference)
import jax
import jax.numpy as jnp
from jax.experimental import pallas as pl


def kernel(x, edge_index, edge_attr, W0, b0, W1, b1, W2, b2):
    raise NotImplementedError("write your pallas kernel here")


# speedup vs baseline: 38.2486x; 38.2486x over previous
import functools

import jax
import jax.numpy as jnp
from jax import lax
from jax.experimental import pallas as pl
from jax.experimental.pallas import tpu as pltpu
from jax.experimental.pallas import tpu_sc as plsc

N = 10000
E = 320000
D = 32
DEG_W = 16

NC = 2
NS = 16
NW = NC * NS
EPT = E // NW
CHUNK = 1000
NCHUNK = EPT // CHUNK
RPS = 624
RPS_LAST = N - (NS - 1) * RPS


def _sliced_copy(s, src, dst):
    @pl.when(s < NS - 1)
    def _():
        pltpu.sync_copy(src.at[pl.ds(s * RPS, RPS)],
                        dst.at[pl.ds(s * RPS, RPS)])

    @pl.when(s == NS - 1)
    def _():
        pltpu.sync_copy(src.at[pl.ds((NS - 1) * RPS, RPS_LAST)],
                        dst.at[pl.ds((NS - 1) * RPS, RPS_LAST)])


def _edge_agg_body(y_hbm, src_hbm, dst_hbm, zero_hbm, out_hbm,
                   acc_sh, idx_s, idx_d, rows, sem_g, sem_s):
    c = lax.axis_index("c")
    s = lax.axis_index("s")
    wid = s * NC + c

    _sliced_copy(s, zero_hbm, acc_sh)
    plsc.subcore_barrier()

    def step(i, carry):
        off = wid * EPT + i * CHUNK
        pltpu.sync_copy(src_hbm.at[pl.ds(off, CHUNK)], idx_s)
        pltpu.sync_copy(dst_hbm.at[pl.ds(off, CHUNK)], idx_d)
        pltpu.async_copy(y_hbm.at[idx_s], rows, sem_g).wait()
        pltpu.async_copy(rows, acc_sh.at[idx_d], sem_s, add=True).wait()
        return carry

    lax.fori_loop(0, NCHUNK, step, 0)
    plsc.subcore_barrier()
    _sliced_copy(s, acc_sh, out_hbm.at[c])


@functools.cache
def _get_edge_agg():
    mesh = plsc.VectorSubcoreMesh(core_axis_name="c", subcore_axis_name="s",
                                  num_cores=NC, num_subcores=NS)
    return pl.kernel(
        _edge_agg_body,
        out_type=jax.ShapeDtypeStruct((NC, N, D), jnp.float32),
        mesh=mesh,
        compiler_params=pltpu.CompilerParams(use_tc_tiling_on_sc=False),
        scratch_types=[
            pltpu.VMEM_SHARED((N, D), jnp.float32),
            pltpu.VMEM((CHUNK,), jnp.int32),
            pltpu.VMEM((CHUNK,), jnp.int32),
            pltpu.VMEM((CHUNK, D), jnp.float32),
            pltpu.SemaphoreType.DMA,
            pltpu.SemaphoreType.DMA,
        ],
    )


def _deg_body(dst_hbm, ones_hbm, zero_hbm, out_hbm,
              deg_sh, idx_d, ones_v, sem_s):
    c = lax.axis_index("c")
    s = lax.axis_index("s")
    wid = s * NC + c

    _sliced_copy(s, zero_hbm, deg_sh)
    pltpu.sync_copy(ones_hbm, ones_v)
    plsc.subcore_barrier()

    def step(i, carry):
        off = wid * EPT + i * CHUNK
        pltpu.sync_copy(dst_hbm.at[pl.ds(off, CHUNK)], idx_d)
        pltpu.async_copy(ones_v, deg_sh.at[idx_d], sem_s, add=True).wait()
        return carry

    lax.fori_loop(0, NCHUNK, step, 0)
    plsc.subcore_barrier()
    _sliced_copy(s, deg_sh, out_hbm.at[c])


@functools.cache
def _get_deg():
    mesh = plsc.VectorSubcoreMesh(core_axis_name="c", subcore_axis_name="s",
                                  num_cores=NC, num_subcores=NS)
    return pl.kernel(
        _deg_body,
        out_type=jax.ShapeDtypeStruct((NC, N, DEG_W), jnp.float32),
        mesh=mesh,
        compiler_params=pltpu.CompilerParams(use_tc_tiling_on_sc=False),
        scratch_types=[
            pltpu.VMEM_SHARED((N, DEG_W), jnp.float32),
            pltpu.VMEM((CHUNK,), jnp.int32),
            pltpu.VMEM((CHUNK, DEG_W), jnp.float32),
            pltpu.SemaphoreType.DMA,
        ],
    )


def _tc0_body(x_ref, w_ref, degp_ref, y_ref, dinv_ref):
    deg = degp_ref[0, :, 0:1] + degp_ref[1, :, 0:1] + 1.0
    dinv = lax.rsqrt(deg)
    y = jnp.dot(x_ref[...], w_ref[...], preferred_element_type=jnp.float32)
    y_ref[...] = y * dinv
    dinv_ref[...] = dinv


def _tc0(x, w0, degp):
    return pl.pallas_call(
        _tc0_body,
        out_shape=(jax.ShapeDtypeStruct((N, D), jnp.float32),
                   jax.ShapeDtypeStruct((N, 1), jnp.float32)),
    )(x, w0, degp)


def _tc_mid_body(y_ref, accp_ref, dinv_ref, b_ref, w_ref, o_ref):
    dinv = dinv_ref[...]
    h = (y_ref[...] + accp_ref[0] + accp_ref[1]) * dinv + b_ref[...]
    z = jnp.where(h >= 0.0, h, 0.01 * h)
    o_ref[...] = jnp.dot(z, w_ref[...],
                         preferred_element_type=jnp.float32) * dinv


def _tc_mid(y, accp, dinv, b, w):
    return pl.pallas_call(
        _tc_mid_body,
        out_shape=jax.ShapeDtypeStruct((N, D), jnp.float32),
    )(y, accp, dinv, b, w)


def _tc_fin_body(y_ref, accp_ref, dinv_ref, b_ref, o_ref):
    o_ref[...] = ((y_ref[...] + accp_ref[0] + accp_ref[1]) * dinv_ref[...]
                  + b_ref[...])


def _tc_fin(y, accp, dinv, b):
    return pl.pallas_call(
        _tc_fin_body,
        out_shape=jax.ShapeDtypeStruct((N, D), jnp.float32),
    )(y, accp, dinv, b)


def kernel(x, edge_index, edge_attr, W0, b0, W1, b1, W2, b2):
    src = edge_index[0]
    dst = edge_index[1]
    zero_nd = jnp.zeros((N, D), jnp.float32)
    zero_nw = jnp.zeros((N, DEG_W), jnp.float32)
    ones_cw = jnp.ones((CHUNK, DEG_W), jnp.float32)

    deg_fn = _get_deg()
    agg_fn = _get_edge_agg()
    degp = deg_fn(dst, ones_cw, zero_nw)
    y0, dinv = _tc0(x, W0, degp)
    acc0 = agg_fn(y0, src, dst, zero_nd)
    y1 = _tc_mid(y0, acc0, dinv, b0.reshape(1, D), W1)
    acc1 = agg_fn(y1, src, dst, zero_nd)
    y2 = _tc_mid(y1, acc1, dinv, b1.reshape(1, D), W2)
    acc2 = agg_fn(y2, src, dst, zero_nd)
    return _tc_fin(y2, acc2, dinv, b2.reshape(1, D))


# speedup vs baseline: 47.5113x; 1.2422x over previous
import functools

import jax
import jax.numpy as jnp
from jax import lax
from jax.experimental import pallas as pl
from jax.experimental.pallas import tpu as pltpu
from jax.experimental.pallas import tpu_sc as plsc

N = 10000
E = 320000
D = 32
DEG_W = 16

NC = 2
NS = 16
NW = NC * NS
EPT = E // NW
CHUNK = 1000
NCHUNK = EPT // CHUNK
RPS = 624
RPS_LAST = N - (NS - 1) * RPS


def _sliced_copy(s, src, dst):
    @pl.when(s < NS - 1)
    def _():
        pltpu.sync_copy(src.at[pl.ds(s * RPS, RPS)],
                        dst.at[pl.ds(s * RPS, RPS)])

    @pl.when(s == NS - 1)
    def _():
        pltpu.sync_copy(src.at[pl.ds((NS - 1) * RPS, RPS_LAST)],
                        dst.at[pl.ds((NS - 1) * RPS, RPS_LAST)])


def _edge_agg_body(y_hbm, src_hbm, dst_hbm, zero_hbm, out_hbm, acc_sh,
                   idx_s, idx_d, rows0, rows1,
                   sem_i, sem_g0, sem_g1, sem_s0, sem_s1):
    c = lax.axis_index("c")
    s = lax.axis_index("s")
    wid = s * NC + c

    rows = (rows0, rows1)
    sem_g = (sem_g0, sem_g1)
    sem_s = (sem_s0, sem_s1)

    ih_s = pltpu.async_copy(src_hbm.at[wid], idx_s, sem_i)
    ih_d = pltpu.async_copy(dst_hbm.at[wid], idx_d, sem_i)
    _sliced_copy(s, zero_hbm, acc_sh)
    ih_s.wait()
    ih_d.wait()
    plsc.subcore_barrier()

    gh = [None] * NCHUNK
    sh = [None] * NCHUNK
    gh[0] = pltpu.async_copy(y_hbm.at[idx_s.at[0]], rows[0], sem_g[0])
    for i in range(NCHUNK):
        b = i & 1
        gh[i].wait()
        sh[i] = pltpu.async_copy(rows[b], acc_sh.at[idx_d.at[i]], sem_s[b],
                                 add=True)
        if i + 1 < NCHUNK:
            if i >= 1:
                sh[i - 1].wait()
            gh[i + 1] = pltpu.async_copy(y_hbm.at[idx_s.at[i + 1]],
                                         rows[1 - b], sem_g[1 - b])
    sh[NCHUNK - 1].wait()

    plsc.subcore_barrier()
    _sliced_copy(s, acc_sh, out_hbm.at[c])


@functools.cache
def _get_edge_agg():
    mesh = plsc.VectorSubcoreMesh(core_axis_name="c", subcore_axis_name="s",
                                  num_cores=NC, num_subcores=NS)
    return pl.kernel(
        _edge_agg_body,
        out_type=jax.ShapeDtypeStruct((NC, N, D), jnp.float32),
        mesh=mesh,
        compiler_params=pltpu.CompilerParams(use_tc_tiling_on_sc=False),
        scratch_types=[
            pltpu.VMEM_SHARED((N, D), jnp.float32),
            pltpu.VMEM((NCHUNK, CHUNK), jnp.int32),
            pltpu.VMEM((NCHUNK, CHUNK), jnp.int32),
            pltpu.VMEM((CHUNK, D), jnp.float32),
            pltpu.VMEM((CHUNK, D), jnp.float32),
            pltpu.SemaphoreType.DMA,
            pltpu.SemaphoreType.DMA,
            pltpu.SemaphoreType.DMA,
            pltpu.SemaphoreType.DMA,
            pltpu.SemaphoreType.DMA,
        ],
    )


def _deg_body(dst_hbm, ones_hbm, zero_hbm, out_hbm,
              deg_sh, idx_d, ones_v, sem_s):
    c = lax.axis_index("c")
    s = lax.axis_index("s")
    wid = s * NC + c

    _sliced_copy(s, zero_hbm, deg_sh)
    pltpu.sync_copy(ones_hbm, ones_v)
    plsc.subcore_barrier()

    def step(i, carry):
        off = wid * EPT + i * CHUNK
        pltpu.sync_copy(dst_hbm.at[pl.ds(off, CHUNK)], idx_d)
        pltpu.async_copy(ones_v, deg_sh.at[idx_d], sem_s, add=True).wait()
        return carry

    lax.fori_loop(0, NCHUNK, step, 0)
    plsc.subcore_barrier()
    _sliced_copy(s, deg_sh, out_hbm.at[c])


@functools.cache
def _get_deg():
    mesh = plsc.VectorSubcoreMesh(core_axis_name="c", subcore_axis_name="s",
                                  num_cores=NC, num_subcores=NS)
    return pl.kernel(
        _deg_body,
        out_type=jax.ShapeDtypeStruct((NC, N, DEG_W), jnp.float32),
        mesh=mesh,
        compiler_params=pltpu.CompilerParams(use_tc_tiling_on_sc=False),
        scratch_types=[
            pltpu.VMEM_SHARED((N, DEG_W), jnp.float32),
            pltpu.VMEM((CHUNK,), jnp.int32),
            pltpu.VMEM((CHUNK, DEG_W), jnp.float32),
            pltpu.SemaphoreType.DMA,
        ],
    )


def _tc0_body(x_ref, w_ref, degp_ref, y_ref, dinv_ref):
    deg = degp_ref[0, :, 0:1] + degp_ref[1, :, 0:1] + 1.0
    dinv = lax.rsqrt(deg)
    y = jnp.dot(x_ref[...], w_ref[...], preferred_element_type=jnp.float32)
    y_ref[...] = y * dinv
    dinv_ref[...] = dinv


def _tc0(x, w0, degp):
    return pl.pallas_call(
        _tc0_body,
        out_shape=(jax.ShapeDtypeStruct((N, D), jnp.float32),
                   jax.ShapeDtypeStruct((N, 1), jnp.float32)),
    )(x, w0, degp)


def _tc_mid_body(y_ref, accp_ref, dinv_ref, b_ref, w_ref, o_ref):
    dinv = dinv_ref[...]
    h = (y_ref[...] + accp_ref[0] + accp_ref[1]) * dinv + b_ref[...]
    z = jnp.where(h >= 0.0, h, 0.01 * h)
    o_ref[...] = jnp.dot(z, w_ref[...],
                         preferred_element_type=jnp.float32) * dinv


def _tc_mid(y, accp, dinv, b, w):
    return pl.pallas_call(
        _tc_mid_body,
        out_shape=jax.ShapeDtypeStruct((N, D), jnp.float32),
    )(y, accp, dinv, b, w)


def _tc_fin_body(y_ref, accp_ref, dinv_ref, b_ref, o_ref):
    o_ref[...] = ((y_ref[...] + accp_ref[0] + accp_ref[1]) * dinv_ref[...]
                  + b_ref[...])


def _tc_fin(y, accp, dinv, b):
    return pl.pallas_call(
        _tc_fin_body,
        out_shape=jax.ShapeDtypeStruct((N, D), jnp.float32),
    )(y, accp, dinv, b)


def kernel(x, edge_index, edge_attr, W0, b0, W1, b1, W2, b2):
    src = edge_index[0]
    dst = edge_index[1]
    zero_nd = jnp.zeros((N, D), jnp.float32)
    zero_nw = jnp.zeros((N, DEG_W), jnp.float32)
    ones_cw = jnp.ones((CHUNK, DEG_W), jnp.float32)

    deg_fn = _get_deg()
    agg_fn = _get_edge_agg()
    src3 = src.reshape(NW, NCHUNK, CHUNK)
    dst3 = dst.reshape(NW, NCHUNK, CHUNK)
    degp = deg_fn(dst, ones_cw, zero_nw)
    y0, dinv = _tc0(x, W0, degp)
    acc0 = agg_fn(y0, src3, dst3, zero_nd)
    y1 = _tc_mid(y0, acc0, dinv, b0.reshape(1, D), W1)
    acc1 = agg_fn(y1, src3, dst3, zero_nd)
    y2 = _tc_mid(y1, acc1, dinv, b1.reshape(1, D), W2)
    acc2 = agg_fn(y2, src3, dst3, zero_nd)
    return _tc_fin(y2, acc2, dinv, b2.reshape(1, D))


# speedup vs baseline: 52.1889x; 1.0985x over previous
import functools

import jax
import jax.numpy as jnp
from jax import lax
from jax.experimental import pallas as pl
from jax.experimental.pallas import tpu as pltpu
from jax.experimental.pallas import tpu_sc as plsc

N = 10000
E = 320000
D = 32
DEG_W = 8

NC = 2
NS = 16
NW = NC * NS
EPT = E // NW
CHUNK = 400
NCHUNK = EPT // CHUNK
RPS = 624
RPS_LAST = N - (NS - 1) * RPS


def _sliced_copy(s, src, dst):
    @pl.when(s < NS - 1)
    def _():
        pltpu.sync_copy(src.at[pl.ds(s * RPS, RPS)],
                        dst.at[pl.ds(s * RPS, RPS)])

    @pl.when(s == NS - 1)
    def _():
        pltpu.sync_copy(src.at[pl.ds((NS - 1) * RPS, RPS_LAST)],
                        dst.at[pl.ds((NS - 1) * RPS, RPS_LAST)])


def _edge_agg_body(y_hbm, ei_hbm, zero_hbm, out_hbm, acc_sh,
                   idx_s, idx_d, rows0, rows1, rows2,
                   sem_i, sem_g0, sem_g1, sem_g2, sem_s0, sem_s1, sem_s2):
    c = lax.axis_index("c")
    s = lax.axis_index("s")
    wid = s * NC + c
    ebase = wid * EPT

    rows = (rows0, rows1, rows2)
    sem_g = (sem_g0, sem_g1, sem_g2)
    sem_s = (sem_s0, sem_s1, sem_s2)

    ih_s = [pltpu.async_copy(
        ei_hbm.at[0, pl.ds(ebase + i * CHUNK, CHUNK)], idx_s.at[i], sem_i)
        for i in range(NCHUNK)]
    ih_d = [pltpu.async_copy(
        ei_hbm.at[1, pl.ds(ebase + i * CHUNK, CHUNK)], idx_d.at[i], sem_i)
        for i in range(NCHUNK)]
    _sliced_copy(s, zero_hbm, acc_sh)
    for h in ih_s:
        h.wait()
    for h in ih_d:
        h.wait()
    plsc.subcore_barrier()

    gh = [None] * NCHUNK
    sh = [None] * NCHUNK
    for i in range(min(2, NCHUNK)):
        gh[i] = pltpu.async_copy(y_hbm.at[idx_s.at[i]], rows[i % 3],
                                 sem_g[i % 3])
    for i in range(NCHUNK):
        b = i % 3
        gh[i].wait()
        sh[i] = pltpu.async_copy(rows[b], acc_sh.at[idx_d.at[i]], sem_s[b],
                                 add=True)
        if i >= 1:
            sh[i - 1].wait()
        j = i + 2
        if j < NCHUNK:
            gh[j] = pltpu.async_copy(y_hbm.at[idx_s.at[j]], rows[j % 3],
                                     sem_g[j % 3])
    sh[NCHUNK - 1].wait()

    plsc.subcore_barrier()
    _sliced_copy(s, acc_sh, out_hbm.at[c])


@functools.cache
def _get_edge_agg():
    mesh = plsc.VectorSubcoreMesh(core_axis_name="c", subcore_axis_name="s",
                                  num_cores=NC, num_subcores=NS)
    return pl.kernel(
        _edge_agg_body,
        out_type=jax.ShapeDtypeStruct((NC, N, D), jnp.float32),
        mesh=mesh,
        compiler_params=pltpu.CompilerParams(use_tc_tiling_on_sc=False),
        scratch_types=[
            pltpu.VMEM_SHARED((N, D), jnp.float32),
            pltpu.VMEM((NCHUNK, CHUNK), jnp.int32),
            pltpu.VMEM((NCHUNK, CHUNK), jnp.int32),
            pltpu.VMEM((CHUNK, D), jnp.float32),
            pltpu.VMEM((CHUNK, D), jnp.float32),
            pltpu.VMEM((CHUNK, D), jnp.float32),
            pltpu.SemaphoreType.DMA,
            pltpu.SemaphoreType.DMA,
            pltpu.SemaphoreType.DMA,
            pltpu.SemaphoreType.DMA,
            pltpu.SemaphoreType.DMA,
            pltpu.SemaphoreType.DMA,
            pltpu.SemaphoreType.DMA,
        ],
    )


def _deg_body(ei_hbm, ones_hbm, zero_hbm, out_hbm,
              deg_sh, idx_d, ones_v, sem_i, sem_s):
    c = lax.axis_index("c")
    s = lax.axis_index("s")
    wid = s * NC + c
    ebase = wid * EPT

    ih = [pltpu.async_copy(
        ei_hbm.at[1, pl.ds(ebase + i * CHUNK, CHUNK)], idx_d.at[i], sem_i)
        for i in range(NCHUNK)]
    oh = pltpu.async_copy(ones_hbm, ones_v, sem_i)
    _sliced_copy(s, zero_hbm, deg_sh)
    for h in ih:
        h.wait()
    oh.wait()
    plsc.subcore_barrier()

    sh = [pltpu.async_copy(ones_v, deg_sh.at[idx_d.at[i]], sem_s, add=True)
          for i in range(NCHUNK)]
    for h in sh:
        h.wait()
    plsc.subcore_barrier()
    _sliced_copy(s, deg_sh, out_hbm.at[c])


@functools.cache
def _get_deg():
    mesh = plsc.VectorSubcoreMesh(core_axis_name="c", subcore_axis_name="s",
                                  num_cores=NC, num_subcores=NS)
    return pl.kernel(
        _deg_body,
        out_type=jax.ShapeDtypeStruct((NC, N, DEG_W), jnp.float32),
        mesh=mesh,
        compiler_params=pltpu.CompilerParams(use_tc_tiling_on_sc=False),
        scratch_types=[
            pltpu.VMEM_SHARED((N, DEG_W), jnp.float32),
            pltpu.VMEM((NCHUNK, CHUNK), jnp.int32),
            pltpu.VMEM((CHUNK, DEG_W), jnp.float32),
            pltpu.SemaphoreType.DMA,
            pltpu.SemaphoreType.DMA,
        ],
    )


def _tc0_body(x_ref, w_ref, degp_ref, y_ref, dinv_ref):
    deg = degp_ref[0, :, 0:1] + degp_ref[1, :, 0:1] + 1.0
    dinv = lax.rsqrt(deg)
    y = jnp.dot(x_ref[...], w_ref[...], preferred_element_type=jnp.float32)
    y_ref[...] = y * dinv
    dinv_ref[...] = dinv


def _tc0(x, w0, degp):
    return pl.pallas_call(
        _tc0_body,
        out_shape=(jax.ShapeDtypeStruct((N, D), jnp.float32),
                   jax.ShapeDtypeStruct((N, 1), jnp.float32)),
    )(x, w0, degp)


def _tc_mid_body(y_ref, accp_ref, dinv_ref, b_ref, w_ref, o_ref):
    dinv = dinv_ref[...]
    h = (y_ref[...] + accp_ref[0] + accp_ref[1]) * dinv + b_ref[...]
    z = jnp.where(h >= 0.0, h, 0.01 * h)
    o_ref[...] = jnp.dot(z, w_ref[...],
                         preferred_element_type=jnp.float32) * dinv


def _tc_mid(y, accp, dinv, b, w):
    return pl.pallas_call(
        _tc_mid_body,
        out_shape=jax.ShapeDtypeStruct((N, D), jnp.float32),
    )(y, accp, dinv, b, w)


def _tc_fin_body(y_ref, accp_ref, dinv_ref, b_ref, o_ref):
    o_ref[...] = ((y_ref[...] + accp_ref[0] + accp_ref[1]) * dinv_ref[...]
                  + b_ref[...])


def _tc_fin(y, accp, dinv, b):
    return pl.pallas_call(
        _tc_fin_body,
        out_shape=jax.ShapeDtypeStruct((N, D), jnp.float32),
    )(y, accp, dinv, b)


def kernel(x, edge_index, edge_attr, W0, b0, W1, b1, W2, b2):
    zero_nd = jnp.zeros((N, D), jnp.float32)
    zero_nw = jnp.zeros((N, DEG_W), jnp.float32)
    ones_cw = jnp.ones((CHUNK, DEG_W), jnp.float32)

    deg_fn = _get_deg()
    agg_fn = _get_edge_agg()
    degp = deg_fn(edge_index, ones_cw, zero_nw)
    y0, dinv = _tc0(x, W0, degp)
    acc0 = agg_fn(y0, edge_index, zero_nd)
    y1 = _tc_mid(y0, acc0, dinv, b0.reshape(1, D), W1)
    acc1 = agg_fn(y1, edge_index, zero_nd)
    y2 = _tc_mid(y1, acc1, dinv, b1.reshape(1, D), W2)
    acc2 = agg_fn(y2, edge_index, zero_nd)
    return _tc_fin(y2, acc2, dinv, b2.reshape(1, D))


# speedup vs baseline: 67.5826x; 1.2950x over previous
import functools

import jax
import jax.numpy as jnp
from jax import lax
from jax.experimental import pallas as pl
from jax.experimental.pallas import tpu as pltpu
from jax.experimental.pallas import tpu_sc as plsc

N = 10000
NP = 10240
E = 320000
D = 32
PACK = 128 // D
NPK = NP // PACK
DEG_W = 32

NC = 2
NS = 16
NW = NC * NS
EPT = E // NW
CHUNK = 400
NCHUNK = EPT // CHUNK
RPS = NP // NS


def _sliced_copy(s, src, dst):
    pltpu.sync_copy(src.at[pl.ds(s * RPS, RPS)], dst.at[pl.ds(s * RPS, RPS)])


def _edge_agg_body(y_hbm, ei_hbm, zero_hbm, out_hbm, acc_sh,
                   idx_s, idx_d, rows0, rows1, rows2,
                   sem_i, sem_g0, sem_g1, sem_g2, sem_s0, sem_s1, sem_s2):
    c = lax.axis_index("c")
    s = lax.axis_index("s")
    wid = s * NC + c
    ebase = wid * EPT

    rows = (rows0, rows1, rows2)
    sem_g = (sem_g0, sem_g1, sem_g2)
    sem_s = (sem_s0, sem_s1, sem_s2)

    ih_s = [pltpu.async_copy(
        ei_hbm.at[0, pl.ds(ebase + i * CHUNK, CHUNK)], idx_s.at[i], sem_i)
        for i in range(NCHUNK)]
    ih_d = [pltpu.async_copy(
        ei_hbm.at[1, pl.ds(ebase + i * CHUNK, CHUNK)], idx_d.at[i], sem_i)
        for i in range(NCHUNK)]
    _sliced_copy(s, zero_hbm, acc_sh)
    for h in ih_s:
        h.wait()
    for h in ih_d:
        h.wait()
    plsc.subcore_barrier()

    gh = [None] * NCHUNK
    sh = [None] * NCHUNK
    for i in range(min(2, NCHUNK)):
        gh[i] = pltpu.async_copy(y_hbm.at[idx_s.at[i]], rows[i % 3],
                                 sem_g[i % 3])
    for i in range(NCHUNK):
        b = i % 3
        gh[i].wait()
        sh[i] = pltpu.async_copy(rows[b], acc_sh.at[idx_d.at[i]], sem_s[b],
                                 add=True)
        if i >= 1:
            sh[i - 1].wait()
        j = i + 2
        if j < NCHUNK:
            gh[j] = pltpu.async_copy(y_hbm.at[idx_s.at[j]], rows[j % 3],
                                     sem_g[j % 3])
    sh[NCHUNK - 1].wait()

    plsc.subcore_barrier()
    _sliced_copy(s, acc_sh, out_hbm.at[c])


@functools.cache
def _get_edge_agg():
    mesh = plsc.VectorSubcoreMesh(core_axis_name="c", subcore_axis_name="s",
                                  num_cores=NC, num_subcores=NS)
    return pl.kernel(
        _edge_agg_body,
        out_type=jax.ShapeDtypeStruct((NC, NP, D), jnp.float32),
        mesh=mesh,
        compiler_params=pltpu.CompilerParams(use_tc_tiling_on_sc=False),
        scratch_types=[
            pltpu.VMEM_SHARED((NP, D), jnp.float32),
            pltpu.VMEM((NCHUNK, CHUNK), jnp.int32),
            pltpu.VMEM((NCHUNK, CHUNK), jnp.int32),
            pltpu.VMEM((CHUNK, D), jnp.float32),
            pltpu.VMEM((CHUNK, D), jnp.float32),
            pltpu.VMEM((CHUNK, D), jnp.float32),
            pltpu.SemaphoreType.DMA,
            pltpu.SemaphoreType.DMA,
            pltpu.SemaphoreType.DMA,
            pltpu.SemaphoreType.DMA,
            pltpu.SemaphoreType.DMA,
            pltpu.SemaphoreType.DMA,
            pltpu.SemaphoreType.DMA,
        ],
    )


def _deg_body(ei_hbm, ones_hbm, zero_hbm, out_hbm,
              deg_sh, idx_d, ones_v, sem_i, sem_s):
    c = lax.axis_index("c")
    s = lax.axis_index("s")
    wid = s * NC + c
    ebase = wid * EPT

    ih = [pltpu.async_copy(
        ei_hbm.at[1, pl.ds(ebase + i * CHUNK, CHUNK)], idx_d.at[i], sem_i)
        for i in range(NCHUNK)]
    oh = pltpu.async_copy(ones_hbm, ones_v, sem_i)
    _sliced_copy(s, zero_hbm, deg_sh)
    for h in ih:
        h.wait()
    oh.wait()
    plsc.subcore_barrier()

    sh = [pltpu.async_copy(ones_v, deg_sh.at[idx_d.at[i]], sem_s, add=True)
          for i in range(NCHUNK)]
    for h in sh:
        h.wait()
    plsc.subcore_barrier()
    _sliced_copy(s, deg_sh, out_hbm.at[c])


@functools.cache
def _get_deg():
    mesh = plsc.VectorSubcoreMesh(core_axis_name="c", subcore_axis_name="s",
                                  num_cores=NC, num_subcores=NS)
    return pl.kernel(
        _deg_body,
        out_type=jax.ShapeDtypeStruct((NC, NP, DEG_W), jnp.float32),
        mesh=mesh,
        compiler_params=pltpu.CompilerParams(use_tc_tiling_on_sc=False),
        scratch_types=[
            pltpu.VMEM_SHARED((NP, DEG_W), jnp.float32),
            pltpu.VMEM((NCHUNK, CHUNK), jnp.int32),
            pltpu.VMEM((CHUNK, DEG_W), jnp.float32),
            pltpu.SemaphoreType.DMA,
            pltpu.SemaphoreType.DMA,
        ],
    )


def _tc0_body(x4_ref, w_ref, degp_ref, y_ref, dinv_ref):
    dinv = lax.rsqrt(degp_ref[0] + degp_ref[1] + 1.0)
    xw = jnp.dot(x4_ref[...], w_ref[...], preferred_element_type=jnp.float32)
    y_ref[...] = xw * dinv
    dinv_ref[...] = dinv


def _tc0(x4, w0s, degp_pk):
    return pl.pallas_call(
        _tc0_body,
        out_shape=(jax.ShapeDtypeStruct((NPK, 128), jnp.float32),
                   jax.ShapeDtypeStruct((NPK, 128), jnp.float32)),
    )(x4, w0s, degp_pk)


def _tc_mid_body(y_ref, accp_ref, dinv_ref, b_ref, w_ref, o_ref):
    dinv = dinv_ref[...]
    h = (y_ref[...] + accp_ref[0] + accp_ref[1]) * dinv + b_ref[...]
    z = jnp.where(h >= 0.0, h, 0.01 * h)
    o_ref[...] = jnp.dot(z, w_ref[...],
                         preferred_element_type=jnp.float32) * dinv


def _tc_mid(y_pk, accp_pk, dinv_pk, b128, wbd):
    return pl.pallas_call(
        _tc_mid_body,
        out_shape=jax.ShapeDtypeStruct((NPK, 128), jnp.float32),
    )(y_pk, accp_pk, dinv_pk, b128, wbd)


def _tc_fin_body(y_ref, accp_ref, dinv_ref, b_ref, o_ref):
    o_ref[...] = ((y_ref[...] + accp_ref[0] + accp_ref[1]) * dinv_ref[...]
                  + b_ref[...])


def _tc_fin(y_pk, accp_pk, dinv_pk, b128):
    return pl.pallas_call(
        _tc_fin_body,
        out_shape=jax.ShapeDtypeStruct((NPK, 128), jnp.float32),
    )(y_pk, accp_pk, dinv_pk, b128)


def kernel(x, edge_index, edge_attr, W0, b0, W1, b1, W2, b2):
    zero_nd = jnp.zeros((NP, D), jnp.float32)
    zero_nw = jnp.zeros((NP, DEG_W), jnp.float32)
    ones_cw = jnp.ones((CHUNK, DEG_W), jnp.float32)
    eye4 = jnp.eye(PACK, dtype=jnp.float32)
    w0s = jnp.kron(eye4, W0)
    w1bd = jnp.kron(eye4, W1)
    w2bd = jnp.kron(eye4, W2)
    b0p = jnp.tile(b0, PACK).reshape(1, 128)
    b1p = jnp.tile(b1, PACK).reshape(1, 128)
    b2p = jnp.tile(b2, PACK).reshape(1, 128)
    x4 = jnp.pad(x, ((0, NP - N), (0, 0))).reshape(NPK, PACK * 128)

    deg_fn = _get_deg()
    agg_fn = _get_edge_agg()
    degp = deg_fn(edge_index, ones_cw, zero_nw)
    y0_pk, dinv_pk = _tc0(x4, w0s, degp.reshape(NC, NPK, 128))
    acc0 = agg_fn(y0_pk.reshape(NP, D), edge_index, zero_nd)
    y1_pk = _tc_mid(y0_pk, acc0.reshape(NC, NPK, 128), dinv_pk, b0p, w1bd)
    acc1 = agg_fn(y1_pk.reshape(NP, D), edge_index, zero_nd)
    y2_pk = _tc_mid(y1_pk, acc1.reshape(NC, NPK, 128), dinv_pk, b1p, w2bd)
    acc2 = agg_fn(y2_pk.reshape(NP, D), edge_index, zero_nd)
    out_pk = _tc_fin(y2_pk, acc2.reshape(NC, NPK, 128), dinv_pk, b2p)
    return out_pk.reshape(NP, D)[:N]
